# baseline (speedup 1.0000x reference)
import jax
import jax.numpy as jnp
from jax import lax
from jax.experimental import pallas as pl
from jax.experimental.pallas import tpu as pltpu
from jax.experimental.pallas import tpu_sc as plsc

_SCALES = (0.25, 0.125, 0.0625, 0.03125)
_SIZES = (128, 64, 32, 16)
_LVL_OFF = (0, 16384, 20480, 21504)
_BATCH_STRIDE = 21760

_NC, _NS, _L = 2, 16, 16
_NW = _NC * _NS
_N = 1024
_PPW = _N // _NW
_OH, _OW = 7, 7
_PTS = _OH * _OW
_C = 256
_CC = _C // _L
_CH = 32
_NCHUNK = _PPW * _PTS // _CH


def _roi_body(table, bx0, by0, bx1, by1, bsc, bbase, bwid, out,
              x0v, y0v, x1v, y1v, scv, basev, widv,
              wuv, huv, cxv, cyv, wm1v,
              idx_all, wa_all, wb_all, wc_all, wd_all,
              rows0, rows1, outv0, outv1, sem0, sem1, osem0, osem1):
    cid = lax.axis_index("c")
    sid = lax.axis_index("s")
    wid = sid * _NC + cid
    pbase = wid * _PPW

    pltpu.sync_copy(bx0.at[pl.ds(pbase, _PPW)], x0v)
    pltpu.sync_copy(by0.at[pl.ds(pbase, _PPW)], y0v)
    pltpu.sync_copy(bx1.at[pl.ds(pbase, _PPW)], x1v)
    pltpu.sync_copy(by1.at[pl.ds(pbase, _PPW)], y1v)
    pltpu.sync_copy(bsc.at[pl.ds(pbase, _PPW)], scv)
    pltpu.sync_copy(bbase.at[pl.ds(pbase, _PPW)], basev)
    pltpu.sync_copy(bwid.at[pl.ds(pbase, _PPW)], widv)

    for g in range(_PPW // _L):
        sl = pl.ds(g * _L, _L)
        sc = scv[sl]
        p0 = x0v[sl] * sc
        p2 = x1v[sl] * sc
        q0 = y0v[sl] * sc
        q2 = y1v[sl] * sc
        wu = (p2 - p0) / float(_OW)
        hu = (q2 - q0) / float(_OH)
        wuv[sl] = wu
        huv[sl] = hu
        cxv[sl] = wu * 0.5 + p0
        cyv[sl] = hu * 0.5 + q0
        wm1v[sl] = widv[sl].astype(jnp.float32) - 1.0

    lane = lax.iota(jnp.int32, _L)
    zeros16 = jnp.zeros((_L,), jnp.int32)

    def idx_compute(tsc, ptv, trow):
        for g in range(_CH // _L):
            pt = ptv + (g * _L)
            p = ((pt.astype(jnp.float32) + 0.5) * (1.0 / _PTS)).astype(jnp.int32)
            p = jnp.minimum(p, _PPW - 1)
            ij = pt - p * _PTS
            ii = ((ij.astype(jnp.float32) + 0.5) * (1.0 / _OW)).astype(jnp.int32)
            jj = ij - ii * _OW
            wu = plsc.load_gather(wuv, [p])
            hu = plsc.load_gather(huv, [p])
            cx = plsc.load_gather(cxv, [p])
            cy = plsc.load_gather(cyv, [p])
            wm1 = plsc.load_gather(wm1v, [p])
            bas = plsc.load_gather(basev, [p])
            wdt = plsc.load_gather(widv, [p])
            x = jj.astype(jnp.float32) * wu + cx
            y = ii.astype(jnp.float32) * hu + cy
            x0i = x.astype(jnp.int32)
            y0i = y.astype(jnp.int32)
            wm1i = wdt - 1
            zf = jnp.zeros((_L,), jnp.float32)
            xc = jnp.minimum(jnp.maximum(x, zf), wm1)
            yc = jnp.minimum(jnp.maximum(y, zf), wm1)
            x0c = jnp.minimum(jnp.maximum(x0i, zeros16), wm1i)
            x1c = jnp.minimum(jnp.maximum(x0i + 1, zeros16), wm1i)
            y0c = jnp.minimum(jnp.maximum(y0i, zeros16), wm1i)
            y1c = jnp.minimum(jnp.maximum(y0i + 1, zeros16), wm1i)
            wxa = x1c.astype(jnp.float32) - xc
            wxb = xc - x0c.astype(jnp.float32)
            wya = y1c.astype(jnp.float32) - yc
            wyb = yc - y0c.astype(jnp.float32)
            r0 = bas + y0c * wdt
            r1 = bas + y1c * wdt
            k4 = (g * _L + lane) * 4
            plsc.store_scatter(idx_all, [trow, k4], r0 + x0c)
            plsc.store_scatter(idx_all, [trow, k4 + 1], r1 + x0c)
            plsc.store_scatter(idx_all, [trow, k4 + 2], r0 + x1c)
            plsc.store_scatter(idx_all, [trow, k4 + 3], r1 + x1c)
            sl = pl.ds(tsc * _CH + g * _L, _L)
            wa_all[sl] = wxa * wya
            wb_all[sl] = wxa * wyb
            wc_all[sl] = wxb * wya
            wd_all[sl] = wxb * wyb
        return ptv + _CH, trow + 1

    def fma_chunk(rowsb, t, outv, osem):
        for g in range(_CH // _L):
            wsl = pl.ds(t * _CH + g * _L, _L)
            wag = wa_all[wsl]
            wbg = wb_all[wsl]
            wcg = wc_all[wsl]
            wdg = wd_all[wsl]

            def fma_body(k2, kiv):
                for s in range(2):
                    k = g * _L + k2 * 2 + s
                    ki = kiv + s
                    was = jnp.take_along_axis(wag, ki, axis=0)
                    wbs = jnp.take_along_axis(wbg, ki, axis=0)
                    wcs = jnp.take_along_axis(wcg, ki, axis=0)
                    wds = jnp.take_along_axis(wdg, ki, axis=0)
                    r4 = k * 4
                    for cc in range(_CC):
                        csl = pl.ds(cc * _L, _L)
                        acc = (was * rowsb[r4, csl] + wbs * rowsb[r4 + 1, csl]
                               + wcs * rowsb[r4 + 2, csl]
                               + wds * rowsb[r4 + 3, csl])
                        outv[pl.ds(k * _C + cc * _L, _L)] = acc
                return kiv + 2
            lax.fori_loop(0, _L // 2, fma_body, zeros16)
        pltpu.async_copy(outv, out.at[pl.ds((pbase * _PTS + t * _CH) * _C, _CH * _C)], osem)

    def owait(outv, t, osem):
        pltpu.make_async_copy(
            outv, out.at[pl.ds((pbase * _PTS + t * _CH) * _C, _CH * _C)],
            osem).wait()

    ptv, trw = idx_compute(0, lane, zeros16)
    ptv, trw = idx_compute(1, ptv, trw)
    pltpu.async_copy(table.at[idx_all.at[0]], rows0, sem0)
    def pair_body(u, carry):
        ptv, trw = carry
        t0 = u * 2
        pltpu.async_copy(table.at[idx_all.at[t0 + 1]], rows1, sem1)
        ptv, trw = idx_compute(t0 + 2, ptv, trw)
        pltpu.make_async_copy(table.at[idx_all.at[t0]], rows0, sem0).wait()
        @pl.when(u > 0)
        def _():
            owait(outv0, t0 - 2, osem0)
        fma_chunk(rows0, t0, outv0, osem0)
        pltpu.async_copy(table.at[idx_all.at[t0 + 2]], rows0, sem0)
        tw = jnp.where(t0 + 3 < _NCHUNK, t0 + 3, 0)
        trw = trw - _NCHUNK * (trw >= _NCHUNK).astype(jnp.int32)
        ptv, trw = idx_compute(tw, ptv, trw)
        pltpu.make_async_copy(table.at[idx_all.at[t0 + 1]], rows1, sem1).wait()
        @pl.when(u > 0)
        def _():
            owait(outv1, t0 - 1, osem1)
        fma_chunk(rows1, t0 + 1, outv1, osem1)
        return ptv, trw

    lax.fori_loop(0, (_NCHUNK - 1) // 2, pair_body, (ptv, trw))
    pltpu.make_async_copy(table.at[idx_all.at[_NCHUNK - 1]], rows0, sem0).wait()
    owait(outv0, _NCHUNK - 3, osem0)
    fma_chunk(rows0, _NCHUNK - 1, outv0, osem0)
    owait(outv1, _NCHUNK - 2, osem1)
    owait(outv0, _NCHUNK - 1, osem0)


@jax.jit
def kernel(feat_p2, feat_p3, feat_p4, feat_p5, proposals):
    feats = (feat_p2, feat_p3, feat_p4, feat_p5)
    B = proposals.shape[0]
    cat = jnp.concatenate([f.reshape(B, _C, -1) for f in feats], axis=2)
    table = jnp.swapaxes(cat, 1, 2).reshape(-1, _C)

    boxes = proposals.reshape(-1, 4)
    w = boxes[:, 2] - boxes[:, 0]
    h = boxes[:, 3] - boxes[:, 1]
    lvl = jnp.clip(jnp.floor(2.0 + jnp.log2(jnp.sqrt(w * h) / 224.0)),
                   0, 3).astype(jnp.int32)
    bsc = jnp.take(jnp.array(_SCALES, jnp.float32), lvl)
    bbase = ((jnp.arange(_N, dtype=jnp.int32) // (_N // B)) * _BATCH_STRIDE
             + jnp.take(jnp.array(_LVL_OFF, jnp.int32), lvl))
    bwid = jnp.take(jnp.array(_SIZES, jnp.int32), lvl)

    mesh = plsc.VectorSubcoreMesh(core_axis_name="c", subcore_axis_name="s",
                                  num_cores=_NC, num_subcores=_NS)
    roi = pl.kernel(
        _roi_body,
        out_type=jax.ShapeDtypeStruct((_N * _PTS * _C,), jnp.float32),
        mesh=mesh,
        compiler_params=pltpu.CompilerParams(needs_layout_passes=False),
        scratch_types=[
            pltpu.VMEM((_PPW,), jnp.float32),
            pltpu.VMEM((_PPW,), jnp.float32),
            pltpu.VMEM((_PPW,), jnp.float32),
            pltpu.VMEM((_PPW,), jnp.float32),
            pltpu.VMEM((_PPW,), jnp.float32),
            pltpu.VMEM((_PPW,), jnp.int32),
            pltpu.VMEM((_PPW,), jnp.int32),
            pltpu.VMEM((_PPW,), jnp.float32),
            pltpu.VMEM((_PPW,), jnp.float32),
            pltpu.VMEM((_PPW,), jnp.float32),
            pltpu.VMEM((_PPW,), jnp.float32),
            pltpu.VMEM((_PPW,), jnp.float32),
            pltpu.VMEM((_NCHUNK, _CH * 4), jnp.int32),
            pltpu.VMEM((_NCHUNK * _CH,), jnp.float32),
            pltpu.VMEM((_NCHUNK * _CH,), jnp.float32),
            pltpu.VMEM((_NCHUNK * _CH,), jnp.float32),
            pltpu.VMEM((_NCHUNK * _CH,), jnp.float32),
            pltpu.VMEM((_CH * 4, _C), jnp.float32),
            pltpu.VMEM((_CH * 4, _C), jnp.float32),
            pltpu.VMEM((_CH * _C,), jnp.float32),
            pltpu.VMEM((_CH * _C,), jnp.float32),
            pltpu.SemaphoreType.DMA,
            pltpu.SemaphoreType.DMA,
            pltpu.SemaphoreType.DMA,
            pltpu.SemaphoreType.DMA,
        ],
    )
    flat = roi(table, boxes[:, 0], boxes[:, 1], boxes[:, 2], boxes[:, 3],
               bsc, bbase, bwid)
    return jnp.transpose(flat.reshape(_N, _OH, _OW, _C), (0, 3, 1, 2))


# speedup vs baseline: 1.0021x; 1.0021x over previous
import jax
import jax.numpy as jnp
from jax import lax
from jax.experimental import pallas as pl
from jax.experimental.pallas import tpu as pltpu
from jax.experimental.pallas import tpu_sc as plsc

_SCALES = (0.25, 0.125, 0.0625, 0.03125)
_SIZES = (128, 64, 32, 16)
_LVL_OFF = (0, 16384, 20480, 21504)
_BATCH_STRIDE = 21760

_NC, _NS, _L = 2, 16, 16
_NW = _NC * _NS
_N = 1024
_PPW = _N // _NW
_OH, _OW = 7, 7
_PTS = _OH * _OW
_C = 256
_CC = _C // _L
_CH = 32
_NCHUNK = _PPW * _PTS // _CH


def _roi_body(table, bx0, by0, bx1, by1, bsc, bbase, bwid, out,
              x0v, y0v, x1v, y1v, scv, basev, widv,
              wuv, huv, cxv, cyv, wm1v,
              idx_all, wa_all, wb_all, wc_all, wd_all,
              rows0, rows1, outv0, outv1, sem0, sem1, osem0, osem1):
    cid = lax.axis_index("c")
    sid = lax.axis_index("s")
    wid = sid * _NC + cid
    pbase = wid * _PPW

    pltpu.sync_copy(bx0.at[pl.ds(pbase, _PPW)], x0v)
    pltpu.sync_copy(by0.at[pl.ds(pbase, _PPW)], y0v)
    pltpu.sync_copy(bx1.at[pl.ds(pbase, _PPW)], x1v)
    pltpu.sync_copy(by1.at[pl.ds(pbase, _PPW)], y1v)
    pltpu.sync_copy(bsc.at[pl.ds(pbase, _PPW)], scv)
    pltpu.sync_copy(bbase.at[pl.ds(pbase, _PPW)], basev)
    pltpu.sync_copy(bwid.at[pl.ds(pbase, _PPW)], widv)

    for g in range(_PPW // _L):
        sl = pl.ds(g * _L, _L)
        sc = scv[sl]
        p0 = x0v[sl] * sc
        p2 = x1v[sl] * sc
        q0 = y0v[sl] * sc
        q2 = y1v[sl] * sc
        wu = (p2 - p0) / float(_OW)
        hu = (q2 - q0) / float(_OH)
        wuv[sl] = wu
        huv[sl] = hu
        cxv[sl] = wu * 0.5 + p0
        cyv[sl] = hu * 0.5 + q0
        wm1v[sl] = widv[sl].astype(jnp.float32) - 1.0

    lane = lax.iota(jnp.int32, _L)
    zeros16 = jnp.zeros((_L,), jnp.int32)

    def idx_body(t, carry):
        ptv, trow = carry
        for g in range(_CH // _L):
            pt = ptv + (g * _L)
            p = ((pt.astype(jnp.float32) + 0.5) * (1.0 / _PTS)).astype(jnp.int32)
            ij = pt - p * _PTS
            ii = ((ij.astype(jnp.float32) + 0.5) * (1.0 / _OW)).astype(jnp.int32)
            jj = ij - ii * _OW
            wu = plsc.load_gather(wuv, [p])
            hu = plsc.load_gather(huv, [p])
            cx = plsc.load_gather(cxv, [p])
            cy = plsc.load_gather(cyv, [p])
            wm1 = plsc.load_gather(wm1v, [p])
            bas = plsc.load_gather(basev, [p])
            wdt = plsc.load_gather(widv, [p])
            x = jj.astype(jnp.float32) * wu + cx
            y = ii.astype(jnp.float32) * hu + cy
            x0i = x.astype(jnp.int32)
            y0i = y.astype(jnp.int32)
            wm1i = wdt - 1
            zf = jnp.zeros((_L,), jnp.float32)
            xc = jnp.minimum(jnp.maximum(x, zf), wm1)
            yc = jnp.minimum(jnp.maximum(y, zf), wm1)
            x0c = jnp.minimum(jnp.maximum(x0i, zeros16), wm1i)
            x1c = jnp.minimum(jnp.maximum(x0i + 1, zeros16), wm1i)
            y0c = jnp.minimum(jnp.maximum(y0i, zeros16), wm1i)
            y1c = jnp.minimum(jnp.maximum(y0i + 1, zeros16), wm1i)
            wxa = x1c.astype(jnp.float32) - xc
            wxb = xc - x0c.astype(jnp.float32)
            wya = y1c.astype(jnp.float32) - yc
            wyb = yc - y0c.astype(jnp.float32)
            r0 = bas + y0c * wdt
            r1 = bas + y1c * wdt
            k4 = (g * _L + lane) * 4
            plsc.store_scatter(idx_all, [trow, k4], r0 + x0c)
            plsc.store_scatter(idx_all, [trow, k4 + 1], r1 + x0c)
            plsc.store_scatter(idx_all, [trow, k4 + 2], r0 + x1c)
            plsc.store_scatter(idx_all, [trow, k4 + 3], r1 + x1c)
            sl = pl.ds(t * _CH + g * _L, _L)
            wa_all[sl] = wxa * wya
            wb_all[sl] = wxa * wyb
            wc_all[sl] = wxb * wya
            wd_all[sl] = wxb * wyb
        return ptv + _CH, trow + 1

    lax.fori_loop(0, _NCHUNK, idx_body, (lane, zeros16))

    def fma_chunk(rowsb, t, outv, osem):
        for g in range(_CH // _L):
            wsl = pl.ds(t * _CH + g * _L, _L)
            wag = wa_all[wsl]
            wbg = wb_all[wsl]
            wcg = wc_all[wsl]
            wdg = wd_all[wsl]

            def fma_body(k2, kiv):
                for s in range(2):
                    k = g * _L + k2 * 2 + s
                    ki = kiv + s
                    was = jnp.take_along_axis(wag, ki, axis=0)
                    wbs = jnp.take_along_axis(wbg, ki, axis=0)
                    wcs = jnp.take_along_axis(wcg, ki, axis=0)
                    wds = jnp.take_along_axis(wdg, ki, axis=0)
                    r4 = k * 4
                    for cc in range(_CC):
                        csl = pl.ds(cc * _L, _L)
                        acc = (was * rowsb[r4, csl] + wbs * rowsb[r4 + 1, csl]
                               + wcs * rowsb[r4 + 2, csl]
                               + wds * rowsb[r4 + 3, csl])
                        outv[pl.ds(k * _C + cc * _L, _L)] = acc
                return kiv + 2
            lax.fori_loop(0, _L // 2, fma_body, zeros16)
        pltpu.async_copy(outv, out.at[pl.ds((pbase * _PTS + t * _CH) * _C, _CH * _C)], osem)

    def owait(outv, t, osem):
        pltpu.make_async_copy(
            outv, out.at[pl.ds((pbase * _PTS + t * _CH) * _C, _CH * _C)],
            osem).wait()

    pltpu.async_copy(table.at[idx_all.at[0]], rows0, sem0)
    def pair_body(u, carry):
        t0 = u * 2
        pltpu.async_copy(table.at[idx_all.at[t0 + 1]], rows1, sem1)
        pltpu.make_async_copy(table.at[idx_all.at[t0]], rows0, sem0).wait()
        @pl.when(u > 0)
        def _():
            owait(outv0, t0 - 2, osem0)
        fma_chunk(rows0, t0, outv0, osem0)
        pltpu.async_copy(table.at[idx_all.at[t0 + 2]], rows0, sem0)
        pltpu.make_async_copy(table.at[idx_all.at[t0 + 1]], rows1, sem1).wait()
        @pl.when(u > 0)
        def _():
            owait(outv1, t0 - 1, osem1)
        fma_chunk(rows1, t0 + 1, outv1, osem1)
        return carry

    lax.fori_loop(0, (_NCHUNK - 1) // 2, pair_body, 0)
    pltpu.make_async_copy(table.at[idx_all.at[_NCHUNK - 1]], rows0, sem0).wait()
    owait(outv0, _NCHUNK - 3, osem0)
    fma_chunk(rows0, _NCHUNK - 1, outv0, osem0)
    owait(outv1, _NCHUNK - 2, osem1)
    owait(outv0, _NCHUNK - 1, osem0)


@jax.jit
def kernel(feat_p2, feat_p3, feat_p4, feat_p5, proposals):
    feats = (feat_p2, feat_p3, feat_p4, feat_p5)
    B = proposals.shape[0]
    parts = [jnp.swapaxes(f.reshape(B, _C, -1), 1, 2) for f in feats]
    table = jnp.concatenate(parts, axis=1).reshape(-1, _C)

    boxes = proposals.reshape(-1, 4)
    w = boxes[:, 2] - boxes[:, 0]
    h = boxes[:, 3] - boxes[:, 1]
    lvl = jnp.clip(jnp.floor(2.0 + jnp.log2(jnp.sqrt(w * h) / 224.0)),
                   0, 3).astype(jnp.int32)
    bsc = jnp.take(jnp.array(_SCALES, jnp.float32), lvl)
    bbase = ((jnp.arange(_N, dtype=jnp.int32) // (_N // B)) * _BATCH_STRIDE
             + jnp.take(jnp.array(_LVL_OFF, jnp.int32), lvl))
    bwid = jnp.take(jnp.array(_SIZES, jnp.int32), lvl)

    mesh = plsc.VectorSubcoreMesh(core_axis_name="c", subcore_axis_name="s",
                                  num_cores=_NC, num_subcores=_NS)
    roi = pl.kernel(
        _roi_body,
        out_type=jax.ShapeDtypeStruct((_N * _PTS * _C,), jnp.float32),
        mesh=mesh,
        compiler_params=pltpu.CompilerParams(needs_layout_passes=False),
        scratch_types=[
            pltpu.VMEM((_PPW,), jnp.float32),
            pltpu.VMEM((_PPW,), jnp.float32),
            pltpu.VMEM((_PPW,), jnp.float32),
            pltpu.VMEM((_PPW,), jnp.float32),
            pltpu.VMEM((_PPW,), jnp.float32),
            pltpu.VMEM((_PPW,), jnp.int32),
            pltpu.VMEM((_PPW,), jnp.int32),
            pltpu.VMEM((_PPW,), jnp.float32),
            pltpu.VMEM((_PPW,), jnp.float32),
            pltpu.VMEM((_PPW,), jnp.float32),
            pltpu.VMEM((_PPW,), jnp.float32),
            pltpu.VMEM((_PPW,), jnp.float32),
            pltpu.VMEM((_NCHUNK, _CH * 4), jnp.int32),
            pltpu.VMEM((_NCHUNK * _CH,), jnp.float32),
            pltpu.VMEM((_NCHUNK * _CH,), jnp.float32),
            pltpu.VMEM((_NCHUNK * _CH,), jnp.float32),
            pltpu.VMEM((_NCHUNK * _CH,), jnp.float32),
            pltpu.VMEM((_CH * 4, _C), jnp.float32),
            pltpu.VMEM((_CH * 4, _C), jnp.float32),
            pltpu.VMEM((_CH * _C,), jnp.float32),
            pltpu.VMEM((_CH * _C,), jnp.float32),
            pltpu.SemaphoreType.DMA,
            pltpu.SemaphoreType.DMA,
            pltpu.SemaphoreType.DMA,
            pltpu.SemaphoreType.DMA,
        ],
    )
    flat = roi(table, boxes[:, 0], boxes[:, 1], boxes[:, 2], boxes[:, 3],
               bsc, bbase, bwid)
    return jnp.transpose(flat.reshape(_N, _OH, _OW, _C), (0, 3, 1, 2))


# speedup vs baseline: 1.0031x; 1.0010x over previous
import jax
import jax.numpy as jnp
from jax import lax
from jax.experimental import pallas as pl
from jax.experimental.pallas import tpu as pltpu
from jax.experimental.pallas import tpu_sc as plsc

_SCALES = (0.25, 0.125, 0.0625, 0.03125)
_SIZES = (128, 64, 32, 16)
_LVL_OFF = (0, 16384, 20480, 21504)
_BATCH_STRIDE = 21760

_NC, _NS, _L = 2, 16, 16
_NW = _NC * _NS
_N = 1024
_PPW = _N // _NW
_OH, _OW = 7, 7
_PTS = _OH * _OW
_C = 256
_CC = _C // _L
_CH = 32
_NCHUNK = _PPW * _PTS // _CH


def _roi_body(table, bx0, by0, bx1, by1, bsc, bbase, bwid, out,
              x0v, y0v, x1v, y1v, scv, basev, widv,
              wuv, huv, cxv, cyv, wm1v,
              idx_all, wa_all, wb_all, wc_all, wd_all,
              rows0, rows1, outv0, outv1, sem0, sem1, osem0, osem1):
    cid = lax.axis_index("c")
    sid = lax.axis_index("s")
    wid = sid * _NC + cid
    pbase = wid * _PPW

    pltpu.sync_copy(bx0.at[pl.ds(pbase, _PPW)], x0v)
    pltpu.sync_copy(by0.at[pl.ds(pbase, _PPW)], y0v)
    pltpu.sync_copy(bx1.at[pl.ds(pbase, _PPW)], x1v)
    pltpu.sync_copy(by1.at[pl.ds(pbase, _PPW)], y1v)
    pltpu.sync_copy(bsc.at[pl.ds(pbase, _PPW)], scv)
    pltpu.sync_copy(bbase.at[pl.ds(pbase, _PPW)], basev)
    pltpu.sync_copy(bwid.at[pl.ds(pbase, _PPW)], widv)

    for g in range(_PPW // _L):
        sl = pl.ds(g * _L, _L)
        sc = scv[sl]
        p0 = x0v[sl] * sc
        p2 = x1v[sl] * sc
        q0 = y0v[sl] * sc
        q2 = y1v[sl] * sc
        wu = (p2 - p0) / float(_OW)
        hu = (q2 - q0) / float(_OH)
        wuv[sl] = wu
        huv[sl] = hu
        cxv[sl] = wu * 0.5 + p0
        cyv[sl] = hu * 0.5 + q0
        wm1v[sl] = widv[sl].astype(jnp.float32) - 1.0

    lane = lax.iota(jnp.int32, _L)
    zeros16 = jnp.zeros((_L,), jnp.int32)

    def idx_body(t, carry):
        ptv, trow = carry
        for g in range(_CH // _L):
            pt = ptv + (g * _L)
            p = ((pt.astype(jnp.float32) + 0.5) * (1.0 / _PTS)).astype(jnp.int32)
            ij = pt - p * _PTS
            ii = ((ij.astype(jnp.float32) + 0.5) * (1.0 / _OW)).astype(jnp.int32)
            jj = ij - ii * _OW
            wu = plsc.load_gather(wuv, [p])
            hu = plsc.load_gather(huv, [p])
            cx = plsc.load_gather(cxv, [p])
            cy = plsc.load_gather(cyv, [p])
            wm1 = plsc.load_gather(wm1v, [p])
            bas = plsc.load_gather(basev, [p])
            wdt = plsc.load_gather(widv, [p])
            x = jj.astype(jnp.float32) * wu + cx
            y = ii.astype(jnp.float32) * hu + cy
            x0i = x.astype(jnp.int32)
            y0i = y.astype(jnp.int32)
            wm1i = wdt - 1
            zf = jnp.zeros((_L,), jnp.float32)
            xc = jnp.minimum(jnp.maximum(x, zf), wm1)
            yc = jnp.minimum(jnp.maximum(y, zf), wm1)
            x0c = jnp.minimum(jnp.maximum(x0i, zeros16), wm1i)
            x1c = jnp.minimum(jnp.maximum(x0i + 1, zeros16), wm1i)
            y0c = jnp.minimum(jnp.maximum(y0i, zeros16), wm1i)
            y1c = jnp.minimum(jnp.maximum(y0i + 1, zeros16), wm1i)
            wxa = x1c.astype(jnp.float32) - xc
            wxb = xc - x0c.astype(jnp.float32)
            wya = y1c.astype(jnp.float32) - yc
            wyb = yc - y0c.astype(jnp.float32)
            r0 = bas + y0c * wdt
            r1 = bas + y1c * wdt
            k4 = (g * _L + lane) * 4
            plsc.store_scatter(idx_all, [trow, k4], r0 + x0c)
            plsc.store_scatter(idx_all, [trow, k4 + 1], r1 + x0c)
            plsc.store_scatter(idx_all, [trow, k4 + 2], r0 + x1c)
            plsc.store_scatter(idx_all, [trow, k4 + 3], r1 + x1c)
            sl = pl.ds(t * _CH + g * _L, _L)
            wa_all[sl] = wxa * wya
            wb_all[sl] = wxa * wyb
            wc_all[sl] = wxb * wya
            wd_all[sl] = wxb * wyb
        return ptv + _CH, trow + 1

    lax.fori_loop(0, _NCHUNK, idx_body, (lane, zeros16))

    def fma_chunk(rowsb, t, outv, osem):
        for g in range(_CH // _L):
            wsl = pl.ds(t * _CH + g * _L, _L)
            wag = wa_all[wsl]
            wbg = wb_all[wsl]
            wcg = wc_all[wsl]
            wdg = wd_all[wsl]

            def fma_body(k2, kiv):
                for s in range(2):
                    k = g * _L + k2 * 2 + s
                    ki = kiv + s
                    was = jnp.take_along_axis(wag, ki, axis=0)
                    wbs = jnp.take_along_axis(wbg, ki, axis=0)
                    wcs = jnp.take_along_axis(wcg, ki, axis=0)
                    wds = jnp.take_along_axis(wdg, ki, axis=0)
                    r4 = k * 4
                    for cc in range(_CC):
                        csl = pl.ds(cc * _L, _L)
                        acc = (was * rowsb[r4, csl] + wbs * rowsb[r4 + 1, csl]
                               + wcs * rowsb[r4 + 2, csl]
                               + wds * rowsb[r4 + 3, csl])
                        outv[pl.ds(k * _C + cc * _L, _L)] = acc
                return kiv + 2
            lax.fori_loop(0, _L // 2, fma_body, zeros16)
        pltpu.async_copy(outv, out.at[pl.ds((pbase * _PTS + t * _CH) * _C, _CH * _C)], osem)

    def owait(outv, t, osem):
        pltpu.make_async_copy(
            outv, out.at[pl.ds((pbase * _PTS + t * _CH) * _C, _CH * _C)],
            osem).wait()

    pltpu.async_copy(table.at[idx_all.at[0]], rows0, sem0)
    def pair_body(u, carry):
        t0 = u * 2
        pltpu.async_copy(table.at[idx_all.at[t0 + 1]], rows1, sem1)
        pltpu.make_async_copy(table.at[idx_all.at[t0]], rows0, sem0).wait()
        @pl.when(u > 0)
        def _():
            owait(outv0, t0 - 2, osem0)
        fma_chunk(rows0, t0, outv0, osem0)
        pltpu.async_copy(table.at[idx_all.at[t0 + 2]], rows0, sem0)
        pltpu.make_async_copy(table.at[idx_all.at[t0 + 1]], rows1, sem1).wait()
        @pl.when(u > 0)
        def _():
            owait(outv1, t0 - 1, osem1)
        fma_chunk(rows1, t0 + 1, outv1, osem1)
        return carry

    lax.fori_loop(0, (_NCHUNK - 1) // 2, pair_body, 0)
    pltpu.make_async_copy(table.at[idx_all.at[_NCHUNK - 1]], rows0, sem0).wait()
    owait(outv0, _NCHUNK - 3, osem0)
    fma_chunk(rows0, _NCHUNK - 1, outv0, osem0)
    owait(outv1, _NCHUNK - 2, osem1)
    owait(outv0, _NCHUNK - 1, osem0)


@jax.jit
def kernel(feat_p2, feat_p3, feat_p4, feat_p5, proposals):
    feats = (feat_p2, feat_p3, feat_p4, feat_p5)
    B = proposals.shape[0]
    cat = jnp.concatenate([f.reshape(B, _C, -1) for f in feats], axis=2)
    table = jnp.swapaxes(cat, 1, 2).reshape(-1, _C)

    boxes = proposals.reshape(-1, 4)
    w = boxes[:, 2] - boxes[:, 0]
    h = boxes[:, 3] - boxes[:, 1]
    lvl = jnp.clip(jnp.floor(2.0 + jnp.log2(jnp.sqrt(w * h) / 224.0)),
                   0, 3).astype(jnp.int32)
    bsc = jnp.take(jnp.array(_SCALES, jnp.float32), lvl)
    bbase = ((jnp.arange(_N, dtype=jnp.int32) // (_N // B)) * _BATCH_STRIDE
             + jnp.take(jnp.array(_LVL_OFF, jnp.int32), lvl))
    bwid = jnp.take(jnp.array(_SIZES, jnp.int32), lvl)

    mesh = plsc.VectorSubcoreMesh(core_axis_name="c", subcore_axis_name="s",
                                  num_cores=_NC, num_subcores=_NS)
    roi = pl.kernel(
        _roi_body,
        out_type=jax.ShapeDtypeStruct((_N * _PTS * _C,), jnp.float32),
        mesh=mesh,
        compiler_params=pltpu.CompilerParams(needs_layout_passes=False),
        scratch_types=[
            pltpu.VMEM((_PPW,), jnp.float32),
            pltpu.VMEM((_PPW,), jnp.float32),
            pltpu.VMEM((_PPW,), jnp.float32),
            pltpu.VMEM((_PPW,), jnp.float32),
            pltpu.VMEM((_PPW,), jnp.float32),
            pltpu.VMEM((_PPW,), jnp.int32),
            pltpu.VMEM((_PPW,), jnp.int32),
            pltpu.VMEM((_PPW,), jnp.float32),
            pltpu.VMEM((_PPW,), jnp.float32),
            pltpu.VMEM((_PPW,), jnp.float32),
            pltpu.VMEM((_PPW,), jnp.float32),
            pltpu.VMEM((_PPW,), jnp.float32),
            pltpu.VMEM((_NCHUNK, _CH * 4), jnp.int32),
            pltpu.VMEM((_NCHUNK * _CH,), jnp.float32),
            pltpu.VMEM((_NCHUNK * _CH,), jnp.float32),
            pltpu.VMEM((_NCHUNK * _CH,), jnp.float32),
            pltpu.VMEM((_NCHUNK * _CH,), jnp.float32),
            pltpu.VMEM((_CH * 4, _C), jnp.float32),
            pltpu.VMEM((_CH * 4, _C), jnp.float32),
            pltpu.VMEM((_CH * _C,), jnp.float32),
            pltpu.VMEM((_CH * _C,), jnp.float32),
            pltpu.SemaphoreType.DMA,
            pltpu.SemaphoreType.DMA,
            pltpu.SemaphoreType.DMA,
            pltpu.SemaphoreType.DMA,
        ],
    )
    flat = roi(table, boxes[:, 0], boxes[:, 1], boxes[:, 2], boxes[:, 3],
               bsc, bbase, bwid)
    return jnp.transpose(flat.reshape(_N, _OH, _OW, _C), (0, 3, 1, 2))
